# baseline (speedup 1.0000x reference)
import functools

import jax
import jax.numpy as jnp
import numpy as np
from jax import lax
from jax.experimental import pallas as pl
from jax.experimental.pallas import tpu as pltpu
from jax.experimental.pallas import tpu_sc as plsc

BINS_N = 10
MU_C = 0.02
MU2_C = MU_C * MU_C
MMT_C = 0.7
ALPHA_C = 0.25
EPS_C = 1e-5

B_N, H_N, W_N, C_N = 64, 256, 256, 5
N_PIX = B_N * H_N * W_N

NC, NS, LANES = 2, 16, 16
NW = NC * NS
PIX_PER_W = N_PIX // NW
CHUNK_PIX = 4096
N_CHUNKS = PIX_PER_W // CHUNK_PIX
CHUNK_F32 = CHUNK_PIX * C_N
GROUPS = CHUNK_PIX // LANES
OUT_ROW = 512

RSQRT_MAGIC = np.int32(0x5F3759DF)

ROW = W_N * C_N
R_BLK = 256
N_ROWS = B_N * H_N
N_BLKS = N_ROWS // R_BLK


def _sc_body(p_hbm, t_hbm, out_hbm, pbuf, tbuf, stage):
    wid = lax.axis_index("s") * NC + lax.axis_index("c")
    base = wid * (PIX_PER_W * C_N)
    lane = lax.iota(jnp.int32, LANES)
    zero16 = jnp.zeros((LANES,), jnp.float32)
    for i in range(OUT_ROW // LANES):
        stage[pl.ds(i * LANES, LANES)] = zero16

    def chunk_body(ch, carry):
        off = base + ch * CHUNK_F32
        pltpu.sync_copy(p_hbm.at[pl.ds(off, CHUNK_F32)], pbuf)
        pltpu.sync_copy(t_hbm.at[pl.ds(off, CHUNK_F32)], tbuf)

        def group_body(g, c2):
            idx0 = (g * LANES + lane) * C_N
            t0 = plsc.load_gather(tbuf, [idx0])
            v_f = jnp.where(t0 > 0.1, 1.0, 0.0).astype(jnp.float32)
            plsc.addupdate(stage.at[pl.ds(320, LANES)], v_f)
            for c in range(1, C_N):
                idx = idx0 + c
                pc = plsc.load_gather(pbuf, [idx])
                tc = plsc.load_gather(tbuf, [idx])
                diff = pc - tc
                s = diff * diff + MU2_C
                bits = plsc.bitcast(s, jnp.int32)
                y = plsc.bitcast(RSQRT_MAGIC - (bits >> 1), jnp.float32)
                y = y * (1.5 - 0.5 * s * y * y)
                y = y * (1.5 - 0.5 * s * y * y)
                y = y * (1.5 - 0.5 * s * y * y)
                loss = s * y - MU_C
                gnorm = jnp.abs(diff) * y
                k = jnp.minimum((gnorm * 10.0).astype(jnp.int32), 9)
                addr = k * LANES + lane
                plsc.addupdate_scatter(stage, [addr], v_f)
                plsc.addupdate_scatter(stage, [addr + 160], v_f * loss)
            return c2

        return lax.fori_loop(0, GROUPS, group_body, carry)

    lax.fori_loop(0, N_CHUNKS, chunk_body, jnp.int32(0))
    pltpu.sync_copy(stage, out_hbm.at[pl.ds(wid * OUT_ROW, OUT_ROW)])


def _sc_pass(p_flat, t_flat):
    mesh = plsc.VectorSubcoreMesh(
        core_axis_name="c", subcore_axis_name="s",
        num_cores=NC, num_subcores=NS,
    )
    f = functools.partial(
        pl.kernel,
        out_type=jax.ShapeDtypeStruct((NW * OUT_ROW,), jnp.float32),
        mesh=mesh,
        scratch_types=[
            pltpu.VMEM((CHUNK_F32,), jnp.float32),
            pltpu.VMEM((CHUNK_F32,), jnp.float32),
            pltpu.VMEM((OUT_ROW,), jnp.float32),
        ],
        compiler_params=pltpu.CompilerParams(
            needs_layout_passes=False,
        ),
    )(_sc_body)
    return f(p_flat, t_flat)


def _focal_kernel(p_ref, t_ref, out_ref):
    p = p_ref[...]
    t = t_ref[...]
    col = jax.lax.broadcasted_iota(jnp.int32, p.shape, 1)
    is_cls = (col % C_N) == 0
    u = 2.0 * t - 1.0
    one_m_t = 1.0 - t
    x_t = p * u + one_m_t
    alpha_t = ALPHA_C * u + one_m_t
    om = 1.0 - x_t
    fl = -alpha_t * om * om * jnp.log(x_t + EPS_C)
    focal_part = jnp.sum(jnp.where(is_cls, fl, 0.0))
    lane = jax.lax.broadcasted_iota(jnp.int32, (1, 1, 128), 2)
    out_ref[...] = jnp.where(lane == 0, focal_part, 0.0)


def _focal_pass(p2d, t2d):
    return pl.pallas_call(
        _focal_kernel,
        grid=(N_BLKS,),
        in_specs=[
            pl.BlockSpec((R_BLK, ROW), lambda i: (i, 0)),
            pl.BlockSpec((R_BLK, ROW), lambda i: (i, 0)),
        ],
        out_specs=pl.BlockSpec((1, 1, 128), lambda i: (i, 0, 0)),
        out_shape=jax.ShapeDtypeStruct((N_BLKS, 1, 128), jnp.float32),
        compiler_params=pltpu.CompilerParams(
            dimension_semantics=("arbitrary",),
        ),
    )(p2d, t2d)


@jax.jit
def kernel(preds, targets):
    p_flat = preds.reshape(-1)
    t_flat = targets.reshape(-1)
    sc_out = _sc_pass(p_flat, t_flat).reshape(NW, OUT_ROW)
    focal_sum = _focal_pass(preds.reshape(N_ROWS, ROW),
                            targets.reshape(N_ROWS, ROW)).sum()

    per_w = sc_out
    counts = per_w[:, 0:160].sum(axis=0).reshape(BINS_N, LANES).sum(axis=1)
    lsum = per_w[:, 160:320].sum(axis=0).reshape(BINS_N, LANES).sum(axis=1)
    tot = jnp.maximum(per_w[:, 320:336].sum(), 1.0)

    acc_sum = (1.0 - MMT_C) * counts
    n = (counts > 0).astype(jnp.float32).sum()
    per_bin_w = jnp.where(counts > 0, tot / jnp.maximum(acc_sum, 1e-12), 0.0)
    reg = (lsum * per_bin_w).sum()
    reg = jnp.where(n > 0, reg / jnp.maximum(n, 1.0), reg)
    reg_loss = reg / tot

    cls_loss = focal_sum / (B_N * H_N * W_N)
    total = cls_loss + reg_loss
    return (total,
            jax.lax.stop_gradient(reg_loss),
            jax.lax.stop_gradient(cls_loss))


# speedup vs baseline: 6.9083x; 6.9083x over previous
import functools

import jax
import jax.numpy as jnp
import numpy as np
from jax.experimental import pallas as pl
from jax.experimental.pallas import tpu as pltpu

BINS_N = 10
MU_C = 0.02
MMT_C = 0.7
ALPHA_C = 0.25
EPS_C = 1e-5

ROW = 1280
R_BLK = 256
N_ROWS = 64 * 256
N_BLKS = N_ROWS // R_BLK


def _edge_list():
    e = [float(x) / BINS_N for x in range(BINS_N + 1)]
    e[-1] = 1000.0
    return [np.float32(v) for v in e]


def _ghm_block_kernel(p_ref, t_ref, out_ref):
    p = p_ref[...]
    t = t_ref[...]
    shape = p.shape
    col = jax.lax.broadcasted_iota(jnp.int32, shape, 1)
    is_cls = (col % 5) == 0

    u = 2.0 * t - 1.0
    one_m_t = 1.0 - t
    x_t = p * u + one_m_t
    alpha_t = ALPHA_C * u + one_m_t
    om = 1.0 - x_t
    fl = -alpha_t * om * om * jnp.log(x_t + EPS_C)
    focal_part = jnp.sum(jnp.where(is_cls, fl, 0.0))

    v = jnp.where(is_cls & (t > 0.1), 1.0, 0.0)
    tot_part = jnp.sum(v)
    vb = v
    for d in range(1, 5):
        vb = vb + jnp.roll(v, d, axis=1)
    vloc = jnp.where(is_cls, 0.0, vb)

    diff = p - t
    d2 = diff * diff
    root = jnp.sqrt(d2 + MU_C * MU_C)
    loss = root - MU_C
    g = jnp.abs(diff / root)
    vl = vloc * loss

    edges = _edge_list()
    partials = [focal_part, tot_part]
    s_list = [jnp.sum(vloc)]
    l_list = [jnp.sum(vl)]
    for b in range(1, BINS_N):
        m = g >= edges[b]
        s_list.append(jnp.sum(jnp.where(m, vloc, 0.0)))
        l_list.append(jnp.sum(jnp.where(m, vl, 0.0)))
    partials += s_list + l_list

    lane = jax.lax.broadcasted_iota(jnp.int32, (1, 1, 128), 2)
    acc = jnp.zeros((1, 1, 128), jnp.float32)
    for j, val in enumerate(partials):
        acc = acc + jnp.where(lane == j, val, 0.0)
    out_ref[...] = acc


def _streaming_pass(p2d, t2d):
    grid = (N_BLKS,)
    return pl.pallas_call(
        _ghm_block_kernel,
        grid=grid,
        in_specs=[
            pl.BlockSpec((R_BLK, ROW), lambda i: (i, 0)),
            pl.BlockSpec((R_BLK, ROW), lambda i: (i, 0)),
        ],
        out_specs=pl.BlockSpec((1, 1, 128), lambda i: (i, 0, 0)),
        out_shape=jax.ShapeDtypeStruct((N_BLKS, 1, 128), jnp.float32),
        compiler_params=pltpu.CompilerParams(
            dimension_semantics=("arbitrary",),
        ),
    )(p2d, t2d)


@jax.jit
def kernel(preds, targets):
    B, H, W, C = preds.shape
    p2d = preds.reshape(N_ROWS, ROW)
    t2d = targets.reshape(N_ROWS, ROW)
    parts = _streaming_pass(p2d, t2d).sum(axis=(0, 1))

    focal_sum = parts[0]
    tot = jnp.maximum(parts[1], 1.0)
    S = parts[2:2 + BINS_N]
    L = parts[2 + BINS_N:2 + 2 * BINS_N]
    counts = S - jnp.concatenate([S[1:], jnp.zeros((1,), jnp.float32)])
    lsum = L - jnp.concatenate([L[1:], jnp.zeros((1,), jnp.float32)])

    acc_sum = (1.0 - MMT_C) * counts
    n = (counts > 0).astype(jnp.float32).sum()
    per_bin_w = jnp.where(counts > 0, tot / jnp.maximum(acc_sum, 1e-12), 0.0)
    bin_contrib = lsum * per_bin_w
    reg = bin_contrib.sum()
    reg = jnp.where(n > 0, reg / jnp.maximum(n, 1.0), reg)
    reg_loss = reg / tot

    cls_loss = focal_sum / (B * H * W)
    total = cls_loss + reg_loss
    return (total,
            jax.lax.stop_gradient(reg_loss),
            jax.lax.stop_gradient(cls_loss))
